# baseline (speedup 1.0000x reference)
import functools

import jax
import jax.numpy as jnp
from jax import lax
from jax.experimental import pallas as pl
from jax.experimental.pallas import tpu as pltpu
from jax.experimental.pallas import tpu_sc as plsc

USER_NUM = 1000000
ITEM_NUM = 100000
DIM = 64
BATCH = 4096
NUM_NEG = 300
NEGP = 304
K_NBR = 10
KP = 16
W1 = 1e-06
W2 = 1.0
NEG_WEIGHT = 300.0
GAMMA = 0.0001
LAMBDA = 0.0005

NC = 2
NS = 16
NW = NC * NS
BPW = BATCH // NW


def _sc_body(users_r, pos_r, negp_r, ut_r, it_r, bu2_r, bi2_r, nbrp_r, simp_r,
             nego_r, poso_r, innero_r, simo_r, buo_r, bio_r,
             uidx_v, pidx_v, u_rows, p_rows, nidx_all, rows0, rows1,
             nbr_idx_v, nbr0, nbr1, sim_v, bu_v, bi_v, part0, part1,
             inner_v, pos_v, sem, sem0, sem1, ssem0, ssem1):
    wid = lax.axis_index("s") * NC + lax.axis_index("c")
    base = wid * BPW
    pltpu.sync_copy(users_r.at[pl.ds(base, BPW)], uidx_v)
    pltpu.sync_copy(pos_r.at[pl.ds(base, BPW)], pidx_v)
    pltpu.async_copy(ut_r.at[uidx_v], u_rows, sem).wait()
    pltpu.async_copy(it_r.at[pidx_v], p_rows, sem).wait()
    pltpu.async_copy(bu2_r.at[uidx_v], bu_v, sem).wait()
    pltpu.async_copy(bi2_r.at[pidx_v], bi_v, sem).wait()
    pltpu.async_copy(nbrp_r.at[pidx_v], nbr_idx_v, sem).wait()
    pltpu.async_copy(simp_r.at[pidx_v], sim_v, sem).wait()
    pltpu.sync_copy(bu_v, buo_r.at[pl.ds(base, BPW)])
    pltpu.sync_copy(bi_v, bio_r.at[pl.ds(base, BPW)])
    pltpu.sync_copy(sim_v, simo_r.at[pl.ds(base, BPW)])
    iota16 = lax.iota(jnp.int32, 16)
    zero16 = jnp.zeros((16,), jnp.float32)

    def dotrow(rows_ref, r, u):
        p = rows_ref[r, pl.ds(0, 16)] * u[0]
        for k in range(1, 4):
            p = p + rows_ref[r, pl.ds(k * 16, 16)] * u[k]
        return jnp.sum(p)

    def dot16(rows_ref, row_base, u):
        res = zero16
        for jj in range(16):
            res = jnp.where(iota16 == jj, dotrow(rows_ref, row_base + jj, u),
                            res)
        return res

    pltpu.sync_copy(negp_r.at[pl.ds(base, BPW)], nidx_all)

    def issue(b, rows_buf, nbr_buf, gsem):
        pltpu.async_copy(it_r.at[nidx_all.at[b, pl.ds(0, 128)]],
                         rows_buf.at[pl.ds(0, 128)], gsem)
        pltpu.async_copy(it_r.at[nidx_all.at[b, pl.ds(128, 128)]],
                         rows_buf.at[pl.ds(128, 128)], gsem)
        pltpu.async_copy(it_r.at[nidx_all.at[b, pl.ds(256, 48)]],
                         rows_buf.at[pl.ds(256, 48)], gsem)
        pltpu.async_copy(it_r.at[nbr_idx_v.at[b]], nbr_buf, gsem)

    def drain(rows_buf, nbr_buf, gsem):
        pltpu.make_async_copy(it_r.at[pl.ds(0, NEGP)], rows_buf, gsem).wait()
        pltpu.make_async_copy(it_r.at[pl.ds(0, KP)], nbr_buf, gsem).wait()

    def compute(p, b, rows_buf, nbr_buf, part_buf, ssem, pos_acc):
        row = base + b
        u = [u_rows[b, pl.ds(k * 16, 16)] for k in range(4)]

        @pl.when(p >= 1)
        def _():
            pltpu.make_async_copy(nego_r.at[0], part_buf, ssem).wait()

        def g_body(g, carry2):
            for jj in range(16):
                r = g * 16 + jj
                pv = rows_buf[r, pl.ds(0, 16)] * u[0]
                for k in range(1, 4):
                    pv = pv + rows_buf[r, pl.ds(k * 16, 16)] * u[k]
                part_buf[r, :] = pv
            return carry2

        lax.fori_loop(0, NEGP // 16, g_body, 0)
        pltpu.async_copy(part_buf, nego_r.at[row], ssem)
        inner_v[b, :] = dot16(nbr_buf, 0, u)
        pos_acc = jnp.where(iota16 == (b % 16), dotrow(p_rows, b, u), pos_acc)

        @pl.when(b % 16 == 15)
        def _():
            pos_v[pl.ds(b - 15, 16)] = pos_acc

        return pos_acc

    issue(0, rows0, nbr0, sem0)

    def pair_body(p, pos_acc):
        b0 = 2 * p
        issue(b0 + 1, rows1, nbr1, sem1)
        drain(rows0, nbr0, sem0)
        pos_acc = compute(p, b0, rows0, nbr0, part0, ssem0, pos_acc)

        @pl.when(p < BPW // 2 - 1)
        def _():
            issue(b0 + 2, rows0, nbr0, sem0)

        drain(rows1, nbr1, sem1)
        pos_acc = compute(p, b0 + 1, rows1, nbr1, part1, ssem1, pos_acc)
        return pos_acc

    lax.fori_loop(0, BPW // 2, pair_body, zero16)
    pltpu.make_async_copy(nego_r.at[0], part0, ssem0).wait()
    pltpu.make_async_copy(nego_r.at[0], part1, ssem1).wait()
    pltpu.sync_copy(inner_v, innero_r.at[pl.ds(base, BPW)])
    pltpu.sync_copy(pos_v, poso_r.at[pl.ds(base, BPW)])


_sc_score = functools.partial(
    pl.kernel,
    out_type=[
        jax.ShapeDtypeStruct((BATCH, NEGP, 16), jnp.float32),
        jax.ShapeDtypeStruct((BATCH,), jnp.float32),
        jax.ShapeDtypeStruct((BATCH, KP), jnp.float32),
        jax.ShapeDtypeStruct((BATCH, KP), jnp.float32),
        jax.ShapeDtypeStruct((BATCH, 1), jnp.float32),
        jax.ShapeDtypeStruct((BATCH, 1), jnp.float32),
    ],
    mesh=plsc.VectorSubcoreMesh(core_axis_name="c", subcore_axis_name="s",
                                num_cores=NC, num_subcores=NS),
    compiler_params=pltpu.CompilerParams(needs_layout_passes=False,
                                         use_tc_tiling_on_sc=False),
    scratch_types=[
        pltpu.VMEM((BPW,), jnp.int32),
        pltpu.VMEM((BPW,), jnp.int32),
        pltpu.VMEM((BPW, DIM), jnp.float32),
        pltpu.VMEM((BPW, DIM), jnp.float32),
        pltpu.VMEM((BPW, NEGP), jnp.int32),
        pltpu.VMEM((NEGP, DIM), jnp.float32),
        pltpu.VMEM((NEGP, DIM), jnp.float32),
        pltpu.VMEM((BPW, KP), jnp.int32),
        pltpu.VMEM((KP, DIM), jnp.float32),
        pltpu.VMEM((KP, DIM), jnp.float32),
        pltpu.VMEM((BPW, KP), jnp.float32),
        pltpu.VMEM((BPW, 1), jnp.float32),
        pltpu.VMEM((BPW, 1), jnp.float32),
        pltpu.VMEM((NEGP, 16), jnp.float32),
        pltpu.VMEM((NEGP, 16), jnp.float32),
        pltpu.VMEM((BPW, KP), jnp.float32),
        pltpu.VMEM((BPW,), jnp.float32),
        pltpu.SemaphoreType.DMA,
        pltpu.SemaphoreType.DMA,
        pltpu.SemaphoreType.DMA,
        pltpu.SemaphoreType.DMA,
        pltpu.SemaphoreType.DMA,
    ],
)(_sc_body)


def _sumsq_body(x_ref, o_ref):
    @pl.when(pl.program_id(0) == 0)
    def _():
        o_ref[0, 0] = 0.0

    x = x_ref[...]
    o_ref[0, 0] += jnp.sum(x * x)


def _sumsq(x, block_rows):
    rows = x.shape[0]
    return pl.pallas_call(
        _sumsq_body,
        grid=(rows // block_rows,),
        in_specs=[pl.BlockSpec((block_rows, 128), lambda i: (i, 0))],
        out_specs=pl.BlockSpec(memory_space=pltpu.SMEM),
        out_shape=jax.ShapeDtypeStruct((1, 1), jnp.float32),
    )(x)


def _combine_body(np_ref, ps_ref, bu_ref, bi_ref, inn_ref, sim_ref,
                  nu_ref, ni_ref, o_ref):
    i = pl.program_id(0)
    x = np_ref[...]
    gm = (lax.broadcasted_iota(jnp.int32, (128, 8), 0) // 16
          == lax.broadcasted_iota(jnp.int32, (128, 8), 1)).astype(jnp.float32)
    ns = jax.lax.dot_general(x, gm, (((1,), (0,)), ((), ())),
                             preferred_element_type=jnp.float32)
    flat = (lax.broadcasted_iota(jnp.int32, ns.shape, 0) * 8
            + lax.broadcasted_iota(jnp.int32, ns.shape, 1))
    col = flat % NEGP
    negl = ns + jnp.log1p(jnp.exp(-jnp.abs(ns))) + jnp.maximum(-ns, 0.0)
    negl = jnp.where(col < NUM_NEG, negl, 0.0)
    ps = ps_ref[...]
    pw = W1 + W2 * (bu_ref[...] * bi_ref[...])
    posl = (1.0 + (pw - 1.0)) * (jnp.log1p(jnp.exp(-jnp.abs(ps)))
                                 + jnp.maximum(-ps, 0.0))
    inn = inn_ref[...]
    sim = sim_ref[...]
    li = jnp.sum(-sim * jnp.log(jax.nn.sigmoid(inn)))
    part = (jnp.sum(posl) + (NEG_WEIGHT / NUM_NEG) * jnp.sum(negl)
            + LAMBDA * li)

    @pl.when(i == 0)
    def _():
        o_ref[0, 0] = 0.0

    o_ref[0, 0] += part

    @pl.when(i == pl.num_programs(0) - 1)
    def _():
        o_ref[0, 0] += GAMMA * ((nu_ref[0, 0] + ni_ref[0, 0]) / 2.0)


def _combine(neg_s, pos_r, bu_r, bi_r, inner, sim, n_u, n_i):
    blocks = 16
    rb = BATCH // blocks
    cb = BATCH // blocks // 16
    del cb
    return pl.pallas_call(
        _combine_body,
        grid=(blocks,),
        in_specs=[
            pl.BlockSpec((rb * NEGP * 16 // 128, 128), lambda i: (i, 0)),
            pl.BlockSpec((8, 32), lambda i: (i, 0)),
            pl.BlockSpec((8, 32), lambda i: (i, 0)),
            pl.BlockSpec((8, 32), lambda i: (i, 0)),
            pl.BlockSpec((rb, KP), lambda i: (i, 0)),
            pl.BlockSpec((rb, KP), lambda i: (i, 0)),
            pl.BlockSpec(memory_space=pltpu.SMEM),
            pl.BlockSpec(memory_space=pltpu.SMEM),
        ],
        out_specs=pl.BlockSpec(memory_space=pltpu.SMEM),
        out_shape=jax.ShapeDtypeStruct((1, 1), jnp.float32),
    )(neg_s, pos_r, bu_r, bi_r, inner, sim, n_u, n_i)


def kernel(users, pos_items, neg_items, user_table, item_table, beta_uD,
           beta_iD, ii_neighbor_mat, ii_constraint_mat):
    users32 = users.astype(jnp.int32)
    pos32 = pos_items.astype(jnp.int32)
    negp = jnp.pad(neg_items.astype(jnp.int32), ((0, 0), (0, NEGP - NUM_NEG)))
    nbrp = jnp.pad(ii_neighbor_mat.astype(jnp.int32),
                   ((0, 0), (0, KP - K_NBR)))
    simp = jnp.pad(ii_constraint_mat, ((0, 0), (0, KP - K_NBR)))
    bu2 = beta_uD[:, None]
    bi2 = beta_iD[:, None]

    n_u = _sumsq(user_table.reshape(USER_NUM // 2, 128), 4000)
    n_i = _sumsq(item_table.reshape(ITEM_NUM // 2, 128), 2000)

    neg_s, pos_s, inner, sim, bu, bi = _sc_score(
        users32, pos32, negp, user_table, item_table, bu2, bi2, nbrp, simp)

    out = _combine(neg_s.reshape(BATCH * NEGP * 16 // 128, 128),
                   pos_s.reshape(128, 32), bu.reshape(128, 32),
                   bi.reshape(128, 32), inner, sim, n_u, n_i)
    return out[0, 0]


# speedup vs baseline: 1.0562x; 1.0562x over previous
import functools

import jax
import jax.numpy as jnp
from jax import lax
from jax.experimental import pallas as pl
from jax.experimental.pallas import tpu as pltpu
from jax.experimental.pallas import tpu_sc as plsc

USER_NUM = 1000000
ITEM_NUM = 100000
DIM = 64
BATCH = 4096
NUM_NEG = 300
NEGP = 304
K_NBR = 10
KP = 16
W1 = 1e-06
W2 = 1.0
NEG_WEIGHT = 300.0
GAMMA = 0.0001
LAMBDA = 0.0005

NC = 2
NS = 16
NW = NC * NS
BPW = BATCH // NW


def _sc_body(users_r, pos_r, negp_r, ut_r, it_r, bu2_r, bi2_r, nbrp_r, simp_r,
             nego_r, poso_r, innero_r, simo_r, buo_r, bio_r,
             uidx_v, pidx_v, u_rows, p_rows, nidx_all, rows0, rows1,
             nbr_idx_v, nbr0, nbr1, sim_v, bu_v, bi_v, scores0, scores1,
             inner_v, pos_v, sem, sem0, sem1, ssem0, ssem1):
    wid = lax.axis_index("s") * NC + lax.axis_index("c")
    base = wid * BPW
    pltpu.sync_copy(users_r.at[pl.ds(base, BPW)], uidx_v)
    pltpu.sync_copy(pos_r.at[pl.ds(base, BPW)], pidx_v)
    pltpu.async_copy(ut_r.at[uidx_v], u_rows, sem).wait()
    pltpu.async_copy(it_r.at[pidx_v], p_rows, sem).wait()
    pltpu.async_copy(bu2_r.at[uidx_v], bu_v, sem).wait()
    pltpu.async_copy(bi2_r.at[pidx_v], bi_v, sem).wait()
    pltpu.async_copy(nbrp_r.at[pidx_v], nbr_idx_v, sem).wait()
    pltpu.async_copy(simp_r.at[pidx_v], sim_v, sem).wait()
    pltpu.sync_copy(bu_v, buo_r.at[pl.ds(base, BPW)])
    pltpu.sync_copy(bi_v, bio_r.at[pl.ds(base, BPW)])
    pltpu.sync_copy(sim_v, simo_r.at[pl.ds(base, BPW)])
    iota16 = lax.iota(jnp.int32, 16)
    zero16 = jnp.zeros((16,), jnp.float32)

    def dotrow(rows_ref, r, u):
        p = rows_ref[r, pl.ds(0, 16)] * u[0]
        for k in range(1, 4):
            p = p + rows_ref[r, pl.ds(k * 16, 16)] * u[k]
        return jnp.sum(p)

    def dot16(rows_ref, row_base, u):
        res = zero16
        for jj in range(16):
            res = jnp.where(iota16 == jj, dotrow(rows_ref, row_base + jj, u),
                            res)
        return res

    pltpu.sync_copy(negp_r.at[pl.ds(base, BPW)], nidx_all)

    def issue(b, rows_buf, nbr_buf, gsem):
        pltpu.async_copy(it_r.at[nidx_all.at[b, pl.ds(0, 128)]],
                         rows_buf.at[pl.ds(0, 128)], gsem)
        pltpu.async_copy(it_r.at[nidx_all.at[b, pl.ds(128, 128)]],
                         rows_buf.at[pl.ds(128, 128)], gsem)
        pltpu.async_copy(it_r.at[nidx_all.at[b, pl.ds(256, 48)]],
                         rows_buf.at[pl.ds(256, 48)], gsem)
        pltpu.async_copy(it_r.at[nbr_idx_v.at[b]], nbr_buf, gsem)

    def drain(rows_buf, nbr_buf, gsem):
        pltpu.make_async_copy(it_r.at[pl.ds(0, NEGP)], rows_buf, gsem).wait()
        pltpu.make_async_copy(it_r.at[pl.ds(0, KP)], nbr_buf, gsem).wait()

    def compute(p, b, rows_buf, nbr_buf, scores_buf, ssem, pos_acc):
        row = base + b
        u = [u_rows[b, pl.ds(k * 16, 16)] for k in range(4)]

        @pl.when(p >= 1)
        def _():
            pltpu.make_async_copy(nego_r.at[0], scores_buf, ssem).wait()

        def g_body(g, carry2):
            scores_buf[pl.ds(g * 16, 16)] = dot16(rows_buf, g * 16, u)
            return carry2

        lax.fori_loop(0, NEGP // 16, g_body, 0)
        pltpu.async_copy(scores_buf, nego_r.at[row], ssem)
        inner_v[b, :] = dot16(nbr_buf, 0, u)
        pos_acc = jnp.where(iota16 == (b % 16), dotrow(p_rows, b, u), pos_acc)

        @pl.when(b % 16 == 15)
        def _():
            pos_v[pl.ds(b - 15, 16)] = pos_acc

        return pos_acc

    issue(0, rows0, nbr0, sem0)

    def pair_body(p, pos_acc):
        b0 = 2 * p
        issue(b0 + 1, rows1, nbr1, sem1)
        drain(rows0, nbr0, sem0)
        pos_acc = compute(p, b0, rows0, nbr0, scores0, ssem0, pos_acc)

        @pl.when(p < BPW // 2 - 1)
        def _():
            issue(b0 + 2, rows0, nbr0, sem0)

        drain(rows1, nbr1, sem1)
        pos_acc = compute(p, b0 + 1, rows1, nbr1, scores1, ssem1, pos_acc)
        return pos_acc

    lax.fori_loop(0, BPW // 2, pair_body, zero16)
    pltpu.make_async_copy(nego_r.at[0], scores0, ssem0).wait()
    pltpu.make_async_copy(nego_r.at[0], scores1, ssem1).wait()
    pltpu.sync_copy(inner_v, innero_r.at[pl.ds(base, BPW)])
    pltpu.sync_copy(pos_v, poso_r.at[pl.ds(base, BPW)])


_sc_score = functools.partial(
    pl.kernel,
    out_type=[
        jax.ShapeDtypeStruct((BATCH, NEGP), jnp.float32),
        jax.ShapeDtypeStruct((BATCH,), jnp.float32),
        jax.ShapeDtypeStruct((BATCH, KP), jnp.float32),
        jax.ShapeDtypeStruct((BATCH, KP), jnp.float32),
        jax.ShapeDtypeStruct((BATCH, 1), jnp.float32),
        jax.ShapeDtypeStruct((BATCH, 1), jnp.float32),
    ],
    mesh=plsc.VectorSubcoreMesh(core_axis_name="c", subcore_axis_name="s",
                                num_cores=NC, num_subcores=NS),
    compiler_params=pltpu.CompilerParams(needs_layout_passes=False,
                                         use_tc_tiling_on_sc=False),
    scratch_types=[
        pltpu.VMEM((BPW,), jnp.int32),
        pltpu.VMEM((BPW,), jnp.int32),
        pltpu.VMEM((BPW, DIM), jnp.float32),
        pltpu.VMEM((BPW, DIM), jnp.float32),
        pltpu.VMEM((BPW, NEGP), jnp.int32),
        pltpu.VMEM((NEGP, DIM), jnp.float32),
        pltpu.VMEM((NEGP, DIM), jnp.float32),
        pltpu.VMEM((BPW, KP), jnp.int32),
        pltpu.VMEM((KP, DIM), jnp.float32),
        pltpu.VMEM((KP, DIM), jnp.float32),
        pltpu.VMEM((BPW, KP), jnp.float32),
        pltpu.VMEM((BPW, 1), jnp.float32),
        pltpu.VMEM((BPW, 1), jnp.float32),
        pltpu.VMEM((NEGP,), jnp.float32),
        pltpu.VMEM((NEGP,), jnp.float32),
        pltpu.VMEM((BPW, KP), jnp.float32),
        pltpu.VMEM((BPW,), jnp.float32),
        pltpu.SemaphoreType.DMA,
        pltpu.SemaphoreType.DMA,
        pltpu.SemaphoreType.DMA,
        pltpu.SemaphoreType.DMA,
        pltpu.SemaphoreType.DMA,
    ],
)(_sc_body)


def _sumsq_body(x_ref, o_ref):
    @pl.when(pl.program_id(0) == 0)
    def _():
        o_ref[0, 0] = 0.0

    x = x_ref[...]
    o_ref[0, 0] += jnp.sum(x * x)


def _sumsq(x, block_rows):
    rows = x.shape[0]
    return pl.pallas_call(
        _sumsq_body,
        grid=(rows // block_rows,),
        in_specs=[pl.BlockSpec((block_rows, 128), lambda i: (i, 0))],
        out_specs=pl.BlockSpec(memory_space=pltpu.SMEM),
        out_shape=jax.ShapeDtypeStruct((1, 1), jnp.float32),
    )(x)


def _combine_body(ns_ref, ps_ref, bu_ref, bi_ref, inn_ref, sim_ref,
                  nu_ref, ni_ref, o_ref):
    i = pl.program_id(0)
    ns = ns_ref[...]
    col = lax.broadcasted_iota(jnp.int32, ns.shape, 1)
    negl = ns + jnp.log1p(jnp.exp(-jnp.abs(ns))) + jnp.maximum(-ns, 0.0)
    negl = jnp.where(col < NUM_NEG, negl, 0.0)
    ps = ps_ref[...]
    pw = W1 + W2 * (bu_ref[...] * bi_ref[...])
    posl = (1.0 + (pw - 1.0)) * (jnp.log1p(jnp.exp(-jnp.abs(ps)))
                                 + jnp.maximum(-ps, 0.0))
    inn = inn_ref[...]
    sim = sim_ref[...]
    li = jnp.sum(-sim * jnp.log(jax.nn.sigmoid(inn)))
    part = (jnp.sum(posl) + (NEG_WEIGHT / NUM_NEG) * jnp.sum(negl)
            + LAMBDA * li)

    @pl.when(i == 0)
    def _():
        o_ref[0, 0] = 0.0

    o_ref[0, 0] += part

    @pl.when(i == pl.num_programs(0) - 1)
    def _():
        o_ref[0, 0] += GAMMA * ((nu_ref[0, 0] + ni_ref[0, 0]) / 2.0)


def _combine(neg_s, pos_r, bu_r, bi_r, inner, sim, n_u, n_i):
    blocks = 16
    rb = BATCH // blocks
    cb = BATCH // blocks // 16
    del cb
    return pl.pallas_call(
        _combine_body,
        grid=(blocks,),
        in_specs=[
            pl.BlockSpec((rb, NEGP), lambda i: (i, 0)),
            pl.BlockSpec((8, 32), lambda i: (i, 0)),
            pl.BlockSpec((8, 32), lambda i: (i, 0)),
            pl.BlockSpec((8, 32), lambda i: (i, 0)),
            pl.BlockSpec((rb, KP), lambda i: (i, 0)),
            pl.BlockSpec((rb, KP), lambda i: (i, 0)),
            pl.BlockSpec(memory_space=pltpu.SMEM),
            pl.BlockSpec(memory_space=pltpu.SMEM),
        ],
        out_specs=pl.BlockSpec(memory_space=pltpu.SMEM),
        out_shape=jax.ShapeDtypeStruct((1, 1), jnp.float32),
    )(neg_s, pos_r, bu_r, bi_r, inner, sim, n_u, n_i)


def kernel(users, pos_items, neg_items, user_table, item_table, beta_uD,
           beta_iD, ii_neighbor_mat, ii_constraint_mat):
    users32 = users.astype(jnp.int32)
    pos32 = pos_items.astype(jnp.int32)
    negp = jnp.pad(neg_items.astype(jnp.int32), ((0, 0), (0, NEGP - NUM_NEG)))
    nbrp = jnp.pad(ii_neighbor_mat.astype(jnp.int32),
                   ((0, 0), (0, KP - K_NBR)))
    simp = jnp.pad(ii_constraint_mat, ((0, 0), (0, KP - K_NBR)))
    bu2 = beta_uD[:, None]
    bi2 = beta_iD[:, None]

    n_u = _sumsq(user_table.reshape(USER_NUM // 2, 128), 4000)
    n_i = _sumsq(item_table.reshape(ITEM_NUM // 2, 128), 2000)

    neg_s, pos_s, inner, sim, bu, bi = _sc_score(
        users32, pos32, negp, user_table, item_table, bu2, bi2, nbrp, simp)

    out = _combine(neg_s, pos_s.reshape(128, 32), bu.reshape(128, 32),
                   bi.reshape(128, 32), inner, sim, n_u, n_i)
    return out[0, 0]
